# baseline (speedup 1.0000x reference)
import jax
import jax.numpy as jnp
from jax.experimental import pallas as pl
from jax.experimental.pallas import tpu as pltpu

_B = 8
_N = 512
_L = 4
_H = 64
_VOCAB = 1024
_OUT = 1000
_LAYERS = 2
_STEPS = 4

_G = _B
_W = _G * _N
_OHC = 1024

_F32 = jnp.float32


def _mm(a, b):
    return jax.lax.dot_general(a, b, (((1,), (0,)), ((), ())),
                               preferred_element_type=_F32)


def _mm_t(a, b):
    return jax.lax.dot_general(a, b, (((0,), (0,)), ((), ())),
                               preferred_element_type=_F32)


def _sigmoid_pre(y):
    return 0.5 * jnp.tanh(y) + 0.5


def _fwd_kernel(si_ref, ei_ref, bd_b_ref,
                data_ref, ti_ref, fi_ref,
                ew_ref, wh0_ref, w1_ref, b_ref, wd_ref, wout_ref, bout_ref,
                out_ref):
    iota_col = jax.lax.broadcasted_iota(jnp.int32, (_N, 1), 0)
    iota_row1 = jax.lax.broadcasted_iota(jnp.int32, (1, _N), 1)
    ips = [(iota_col == si_ref[g]).astype(_F32) for g in range(_G)]
    exit_row = jnp.concatenate(
        [iota_row1 == ei_ref[g] for g in range(_G)], axis=1)
    exit_cols = [(iota_col == ei_ref[g]).astype(_F32) for g in range(_G)]

    iota_vcol = jax.lax.broadcasted_iota(jnp.int32, (_VOCAB, _OHC), 0)
    gx0 = []
    for tok in range(_L):
        chunks = []
        for cs in range(0, _W, _OHC):
            drow = data_ref[tok:tok + 1, cs:cs + _OHC]
            oh = (iota_vcol == drow).astype(_F32)
            chunks.append(_mm_t(ew_ref[...], oh))
        gx0.append(jnp.concatenate(chunks, axis=1))

    iota_row = jax.lax.broadcasted_iota(jnp.int32, (_N, _N), 1)
    zeros_nn = jnp.zeros((_N, _N), _F32)
    ones_col = jnp.ones((_N, 1), _F32)
    ones_rows = jnp.ones((8, _N), _F32)

    s_prev = jnp.zeros((4 * _H, _W), _F32)

    for _step in range(_STEPS):
        c0 = s_prev[0 * _H:1 * _H]
        c1 = s_prev[1 * _H:2 * _H]
        h0 = s_prev[2 * _H:3 * _H]
        h1 = s_prev[3 * _H:4 * _H]
        for tok in range(_L):
            g0 = gx0[tok] + _mm_t(wh0_ref[...], h0) + b_ref[0]
            ig = _sigmoid_pre(g0[0 * _H:1 * _H])
            fg = _sigmoid_pre(g0[1 * _H:2 * _H])
            gg = jnp.tanh(g0[2 * _H:3 * _H])
            og = _sigmoid_pre(g0[3 * _H:4 * _H])
            c0 = fg * c0 + ig * gg
            h0 = og * jnp.tanh(c0)
            inp1 = jnp.concatenate([h0, h1], axis=0)
            g1 = _mm_t(w1_ref[...], inp1) + b_ref[1]
            ig = _sigmoid_pre(g1[0 * _H:1 * _H])
            fg = _sigmoid_pre(g1[1 * _H:2 * _H])
            gg = jnp.tanh(g1[2 * _H:3 * _H])
            og = _sigmoid_pre(g1[3 * _H:4 * _H])
            c1 = fg * c1 + ig * gg
            h1 = og * jnp.tanh(c1)
        s_new = jnp.concatenate([c0, c1, h0, h1], axis=0)
        sm = jnp.where(exit_row, s_prev, s_new)

        s_parts = []
        for g in range(_G):
            sm_g = sm[:, g * _N:(g + 1) * _N]
            d = _mm_t(sm_g, wd_ref[...]) + bd_b_ref[0]
            pt = _sigmoid_pre(d) * ips[g]
            pf = ips[g] - pt
            bm = (jnp.where(ti_ref[g] == iota_row, pt, zeros_nn)
                  + jnp.where(fi_ref[g] == iota_row, pf, zeros_nn))
            sm_aug = jnp.concatenate([sm_g, ones_rows], axis=0)
            agg = _mm(sm_aug, bm)
            inv = 1.0 / (agg[4 * _H:4 * _H + 1] + 1e-7)
            s_parts.append(agg[0:4 * _H] * inv)
            ips[g] = _mm_t(bm, ones_col)
        s_prev = jnp.concatenate(s_parts, axis=1)

    es = jnp.concatenate(
        [_mm(s_prev[:, g * _N:(g + 1) * _N], exit_cols[g]) for g in range(_G)],
        axis=1)
    out = _mm_t(es, wout_ref[...]) + bout_ref[...]
    out_ref[...] = out


def kernel(data, true_branch_nodes, false_branch_nodes, start_index, exit_index,
           steps, embed, Wi, Wh, b, W_bd, b_bd, W_out, b_out):
    del steps
    dm = jnp.transpose(data, (1, 0, 2)).reshape(_L, _B * _N)
    ti3 = true_branch_nodes.reshape(_B, _N, 1)
    fi3 = false_branch_nodes.reshape(_B, _N, 1)
    scale = jnp.concatenate([jnp.full((2 * _H,), 0.5, _F32),
                             jnp.ones((_H,), _F32),
                             jnp.full((_H,), 0.5, _F32)])
    ew = embed @ (Wi[0] * scale)
    wh0 = Wh[0] * scale
    w1 = jnp.concatenate([Wi[1], Wh[1]], axis=0) * scale
    b2 = (b * scale).reshape(_LAYERS, 4 * _H, 1)
    wd = ((W_bd[:, 0] - W_bd[:, 1]) * 0.5).reshape(4 * _H, 1)
    bd_b = ((b_bd[0] - b_bd[1]) * 0.5).reshape(1)
    bout2 = b_out.reshape(1, _OUT)
    si = start_index.astype(jnp.int32)
    ei = exit_index.astype(jnp.int32)

    out = pl.pallas_call(
        _fwd_kernel,
        in_specs=[
            pl.BlockSpec(memory_space=pltpu.SMEM),
            pl.BlockSpec(memory_space=pltpu.SMEM),
            pl.BlockSpec(memory_space=pltpu.SMEM),
            pl.BlockSpec((_L, _W), lambda: (0, 0)),
            pl.BlockSpec((_G, _N, 1), lambda: (0, 0, 0)),
            pl.BlockSpec((_G, _N, 1), lambda: (0, 0, 0)),
            pl.BlockSpec((_VOCAB, 4 * _H), lambda: (0, 0)),
            pl.BlockSpec((_H, 4 * _H), lambda: (0, 0)),
            pl.BlockSpec((2 * _H, 4 * _H), lambda: (0, 0)),
            pl.BlockSpec((_LAYERS, 4 * _H, 1), lambda: (0, 0, 0)),
            pl.BlockSpec((4 * _H, 1), lambda: (0, 0)),
            pl.BlockSpec((4 * _H, _OUT), lambda: (0, 0)),
            pl.BlockSpec((1, _OUT), lambda: (0, 0)),
        ],
        out_specs=pl.BlockSpec((_G, _OUT), lambda: (0, 0)),
        out_shape=jax.ShapeDtypeStruct((_G, _OUT), _F32),
    )(si, ei, bd_b, dm, ti3, fi3, ew, wh0, w1, b2, wd, W_out, bout2)
    return out


# speedup vs baseline: 1.0145x; 1.0145x over previous
import jax
import jax.numpy as jnp
from jax.experimental import pallas as pl
from jax.experimental.pallas import tpu as pltpu

_B = 8
_N = 512
_L = 4
_H = 64
_VOCAB = 1024
_OUT = 1000
_LAYERS = 2
_STEPS = 4

_G = _B
_W = _G * _N
_OHC = 1024

_F32 = jnp.float32


def _mm(a, b):
    return jax.lax.dot_general(a, b, (((1,), (0,)), ((), ())),
                               preferred_element_type=_F32)


def _mm_t(a, b):
    return jax.lax.dot_general(a, b, (((0,), (0,)), ((), ())),
                               preferred_element_type=_F32)


def _sigmoid_pre(y):
    return 0.5 * jnp.tanh(y) + 0.5


def _fwd_kernel(si_ref, ei_ref, bd_b_ref,
                data_ref, ti_ref, fi_ref,
                ew_ref, wh0_ref, w1_ref, b_ref, wd_ref, wout_ref, bout_ref,
                out_ref):
    iota_col = jax.lax.broadcasted_iota(jnp.int32, (_N, 1), 0)
    iota_row1 = jax.lax.broadcasted_iota(jnp.int32, (1, _N), 1)
    ips = [(iota_col == si_ref[g]).astype(_F32) for g in range(_G)]
    exit_row = jnp.concatenate(
        [iota_row1 == ei_ref[g] for g in range(_G)], axis=1)
    exit_cols = [(iota_col == ei_ref[g]).astype(_F32) for g in range(_G)]

    iota_vcol = jax.lax.broadcasted_iota(jnp.int32, (_VOCAB, _OHC), 0)
    gx0 = []
    for tok in range(_L):
        chunks = []
        for cs in range(0, _W, _OHC):
            drow = data_ref[tok:tok + 1, cs:cs + _OHC]
            oh = (iota_vcol == drow).astype(_F32)
            chunks.append(_mm_t(ew_ref[...], oh))
        gx0.append(jnp.concatenate(chunks, axis=1))

    iota_row = jax.lax.broadcasted_iota(jnp.int32, (_N, _N), 1)
    t_oh = [(ti_ref[g] == iota_row).astype(_F32) for g in range(_G)]
    f_oh = [(fi_ref[g] == iota_row).astype(_F32) for g in range(_G)]
    ones_col = jnp.ones((_N, 1), _F32)
    ones_rows = jnp.ones((8, _N), _F32)

    s_prev = jnp.zeros((4 * _H, _W), _F32)

    for _step in range(_STEPS):
        c0 = s_prev[0 * _H:1 * _H]
        c1 = s_prev[1 * _H:2 * _H]
        h0 = s_prev[2 * _H:3 * _H]
        h1 = s_prev[3 * _H:4 * _H]
        for tok in range(_L):
            g0 = gx0[tok] + _mm_t(wh0_ref[...], h0) + b_ref[0]
            ig = _sigmoid_pre(g0[0 * _H:1 * _H])
            fg = _sigmoid_pre(g0[1 * _H:2 * _H])
            gg = jnp.tanh(g0[2 * _H:3 * _H])
            og = _sigmoid_pre(g0[3 * _H:4 * _H])
            c0 = fg * c0 + ig * gg
            h0 = og * jnp.tanh(c0)
            inp1 = jnp.concatenate([h0, h1], axis=0)
            g1 = _mm_t(w1_ref[...], inp1) + b_ref[1]
            ig = _sigmoid_pre(g1[0 * _H:1 * _H])
            fg = _sigmoid_pre(g1[1 * _H:2 * _H])
            gg = jnp.tanh(g1[2 * _H:3 * _H])
            og = _sigmoid_pre(g1[3 * _H:4 * _H])
            c1 = fg * c1 + ig * gg
            h1 = og * jnp.tanh(c1)
        s_new = jnp.concatenate([c0, c1, h0, h1], axis=0)
        sm = jnp.where(exit_row, s_prev, s_new)

        s_parts = []
        for g in range(_G):
            sm_g = sm[:, g * _N:(g + 1) * _N]
            d = _mm_t(sm_g, wd_ref[...]) + bd_b_ref[0]
            pt = _sigmoid_pre(d) * ips[g]
            pf = ips[g] - pt
            bm = t_oh[g] * pt + f_oh[g] * pf
            sm_aug = jnp.concatenate([sm_g, ones_rows], axis=0)
            agg = _mm(sm_aug, bm)
            inv = 1.0 / (agg[4 * _H:4 * _H + 1] + 1e-7)
            s_parts.append(agg[0:4 * _H] * inv)
            ips[g] = _mm_t(bm, ones_col)
        s_prev = jnp.concatenate(s_parts, axis=1)

    es = jnp.concatenate(
        [_mm(s_prev[:, g * _N:(g + 1) * _N], exit_cols[g]) for g in range(_G)],
        axis=1)
    out = _mm_t(es, wout_ref[...]) + bout_ref[...]
    out_ref[...] = out


def kernel(data, true_branch_nodes, false_branch_nodes, start_index, exit_index,
           steps, embed, Wi, Wh, b, W_bd, b_bd, W_out, b_out):
    del steps
    dm = jnp.transpose(data, (1, 0, 2)).reshape(_L, _B * _N)
    ti3 = true_branch_nodes.reshape(_B, _N, 1)
    fi3 = false_branch_nodes.reshape(_B, _N, 1)
    scale = jnp.concatenate([jnp.full((2 * _H,), 0.5, _F32),
                             jnp.ones((_H,), _F32),
                             jnp.full((_H,), 0.5, _F32)])
    ew = embed @ (Wi[0] * scale)
    wh0 = Wh[0] * scale
    w1 = jnp.concatenate([Wi[1], Wh[1]], axis=0) * scale
    b2 = (b * scale).reshape(_LAYERS, 4 * _H, 1)
    wd = ((W_bd[:, 0] - W_bd[:, 1]) * 0.5).reshape(4 * _H, 1)
    bd_b = ((b_bd[0] - b_bd[1]) * 0.5).reshape(1)
    bout2 = b_out.reshape(1, _OUT)
    si = start_index.astype(jnp.int32)
    ei = exit_index.astype(jnp.int32)

    out = pl.pallas_call(
        _fwd_kernel,
        in_specs=[
            pl.BlockSpec(memory_space=pltpu.SMEM),
            pl.BlockSpec(memory_space=pltpu.SMEM),
            pl.BlockSpec(memory_space=pltpu.SMEM),
            pl.BlockSpec((_L, _W), lambda: (0, 0)),
            pl.BlockSpec((_G, _N, 1), lambda: (0, 0, 0)),
            pl.BlockSpec((_G, _N, 1), lambda: (0, 0, 0)),
            pl.BlockSpec((_VOCAB, 4 * _H), lambda: (0, 0)),
            pl.BlockSpec((_H, 4 * _H), lambda: (0, 0)),
            pl.BlockSpec((2 * _H, 4 * _H), lambda: (0, 0)),
            pl.BlockSpec((_LAYERS, 4 * _H, 1), lambda: (0, 0, 0)),
            pl.BlockSpec((4 * _H, 1), lambda: (0, 0)),
            pl.BlockSpec((4 * _H, _OUT), lambda: (0, 0)),
            pl.BlockSpec((1, _OUT), lambda: (0, 0)),
        ],
        out_specs=pl.BlockSpec((_G, _OUT), lambda: (0, 0)),
        out_shape=jax.ShapeDtypeStruct((_G, _OUT), _F32),
    )(si, ei, bd_b, dm, ti3, fi3, ew, wh0, w1, b2, wd, W_out, bout2)
    return out


# speedup vs baseline: 1.1137x; 1.0978x over previous
import jax
import jax.numpy as jnp
from jax.experimental import pallas as pl
from jax.experimental.pallas import tpu as pltpu

_B = 8
_N = 512
_L = 4
_H = 64
_VOCAB = 1024
_OUT = 1000
_LAYERS = 2
_STEPS = 4

_G = _B
_W = _G * _N
_OHC = 1024

_F32 = jnp.float32


def _mm(a, b):
    return jax.lax.dot_general(a, b, (((1,), (0,)), ((), ())),
                               preferred_element_type=_F32)


def _mm_t(a, b):
    return jax.lax.dot_general(a, b, (((0,), (0,)), ((), ())),
                               preferred_element_type=_F32)


def _sigmoid_pre(y):
    return 0.5 * jnp.tanh(y) + 0.5


def _fwd_kernel(si_ref, ei_ref, bd_ref,
                data_ref, ti_ref, fi_ref,
                embed_ref, wi_ref, wh_ref, b_ref, wbd_ref, wout_ref, bout_ref,
                out_ref):
    iota_l256 = jax.lax.broadcasted_iota(jnp.int32, (1, 4 * _H), 1)
    scale_row = jnp.where((iota_l256 >= 2 * _H) & (iota_l256 < 3 * _H),
                          jnp.full((1, 4 * _H), 1.0, _F32),
                          jnp.full((1, 4 * _H), 0.5, _F32))
    ones11 = jnp.ones((1, 1), _F32)
    ew = _mm(embed_ref[...], wi_ref[0] * scale_row)
    wh0 = wh_ref[0] * scale_row
    w1 = jnp.concatenate([wi_ref[1], wh_ref[1]], axis=0) * scale_row
    b0 = _mm_t(b_ref[0:1, :] * scale_row, ones11)
    b1 = _mm_t(b_ref[1:2, :] * scale_row, ones11)
    wd = (wbd_ref[:, 0:1] - wbd_ref[:, 1:2]) * 0.5
    bd_b = (bd_ref[0] - bd_ref[1]) * 0.5

    iota_col = jax.lax.broadcasted_iota(jnp.int32, (_N, 1), 0)
    iota_row1 = jax.lax.broadcasted_iota(jnp.int32, (1, _N), 1)
    ips = [(iota_col == si_ref[g]).astype(_F32) for g in range(_G)]
    exit_row = jnp.concatenate(
        [iota_row1 == ei_ref[g] for g in range(_G)], axis=1)
    exit_cols = [(iota_col == ei_ref[g]).astype(_F32) for g in range(_G)]

    iota_vcol = jax.lax.broadcasted_iota(jnp.int32, (_VOCAB, _OHC), 0)
    gx0 = []
    for tok in range(_L):
        chunks = []
        for cs in range(0, _W, _OHC):
            drow = data_ref[tok:tok + 1, cs:cs + _OHC]
            oh = (iota_vcol == drow).astype(_F32)
            chunks.append(_mm_t(ew, oh))
        gx0.append(jnp.concatenate(chunks, axis=1))

    iota_row = jax.lax.broadcasted_iota(jnp.int32, (_N, _N), 1)
    t_oh = [(ti_ref[g] == iota_row).astype(_F32) for g in range(_G)]
    f_oh = [(fi_ref[g] == iota_row).astype(_F32) for g in range(_G)]
    ones_col = jnp.ones((_N, 1), _F32)
    ones_rows = jnp.ones((8, _N), _F32)

    s_prev = jnp.zeros((4 * _H, _W), _F32)

    for _step in range(_STEPS):
        c0 = s_prev[0 * _H:1 * _H]
        c1 = s_prev[1 * _H:2 * _H]
        h0 = s_prev[2 * _H:3 * _H]
        h1 = s_prev[3 * _H:4 * _H]
        for tok in range(_L):
            g0 = gx0[tok] + _mm_t(wh0, h0) + b0
            ig = _sigmoid_pre(g0[0 * _H:1 * _H])
            fg = _sigmoid_pre(g0[1 * _H:2 * _H])
            gg = jnp.tanh(g0[2 * _H:3 * _H])
            og = _sigmoid_pre(g0[3 * _H:4 * _H])
            c0 = fg * c0 + ig * gg
            h0 = og * jnp.tanh(c0)
            inp1 = jnp.concatenate([h0, h1], axis=0)
            g1 = _mm_t(w1, inp1) + b1
            ig = _sigmoid_pre(g1[0 * _H:1 * _H])
            fg = _sigmoid_pre(g1[1 * _H:2 * _H])
            gg = jnp.tanh(g1[2 * _H:3 * _H])
            og = _sigmoid_pre(g1[3 * _H:4 * _H])
            c1 = fg * c1 + ig * gg
            h1 = og * jnp.tanh(c1)
        s_new = jnp.concatenate([c0, c1, h0, h1], axis=0)
        sm = jnp.where(exit_row, s_prev, s_new)

        s_parts = []
        for g in range(_G):
            sm_g = sm[:, g * _N:(g + 1) * _N]
            d = _mm_t(sm_g, wd) + bd_b
            pt = _sigmoid_pre(d) * ips[g]
            pf = ips[g] - pt
            bm = t_oh[g] * pt + f_oh[g] * pf
            sm_aug = jnp.concatenate([sm_g, ones_rows], axis=0)
            agg = _mm(sm_aug, bm)
            inv = 1.0 / (agg[4 * _H:4 * _H + 1] + 1e-7)
            s_parts.append(agg[0:4 * _H] * inv)
            ips[g] = _mm_t(bm, ones_col)
        s_prev = jnp.concatenate(s_parts, axis=1)

    es = jnp.concatenate(
        [_mm(s_prev[:, g * _N:(g + 1) * _N], exit_cols[g]) for g in range(_G)],
        axis=1)
    out = _mm_t(es, wout_ref[...]) + bout_ref[...]
    out_ref[...] = out


def kernel(data, true_branch_nodes, false_branch_nodes, start_index, exit_index,
           steps, embed, Wi, Wh, b, W_bd, b_bd, W_out, b_out):
    del steps
    dm = jnp.transpose(data, (1, 0, 2)).reshape(_L, _B * _N)
    ti3 = true_branch_nodes.reshape(_B, _N, 1)
    fi3 = false_branch_nodes.reshape(_B, _N, 1)
    bout2 = b_out.reshape(1, _OUT)

    out = pl.pallas_call(
        _fwd_kernel,
        in_specs=[
            pl.BlockSpec(memory_space=pltpu.SMEM),
            pl.BlockSpec(memory_space=pltpu.SMEM),
            pl.BlockSpec(memory_space=pltpu.SMEM),
            pl.BlockSpec((_L, _W), lambda: (0, 0)),
            pl.BlockSpec((_G, _N, 1), lambda: (0, 0, 0)),
            pl.BlockSpec((_G, _N, 1), lambda: (0, 0, 0)),
            pl.BlockSpec((_VOCAB, _H), lambda: (0, 0)),
            pl.BlockSpec((_LAYERS, _H, 4 * _H), lambda: (0, 0, 0)),
            pl.BlockSpec((_LAYERS, _H, 4 * _H), lambda: (0, 0, 0)),
            pl.BlockSpec((_LAYERS, 4 * _H), lambda: (0, 0)),
            pl.BlockSpec((4 * _H, 2), lambda: (0, 0)),
            pl.BlockSpec((4 * _H, _OUT), lambda: (0, 0)),
            pl.BlockSpec((1, _OUT), lambda: (0, 0)),
        ],
        out_specs=pl.BlockSpec((_G, _OUT), lambda: (0, 0)),
        out_shape=jax.ShapeDtypeStruct((_G, _OUT), _F32),
    )(start_index, exit_index, b_bd, dm, ti3, fi3, embed, Wi, Wh, b, W_bd,
      W_out, bout2)
    return out


# speedup vs baseline: 1.1434x; 1.0267x over previous
import jax
import jax.numpy as jnp
from jax.experimental import pallas as pl
from jax.experimental.pallas import tpu as pltpu

_B = 8
_N = 512
_L = 4
_H = 64
_VOCAB = 1024
_OUT = 1000
_LAYERS = 2
_STEPS = 4

_G = _B
_W = _G * _N
_OHC = 1024

_F32 = jnp.float32


def _mm(a, b):
    return jax.lax.dot_general(a, b, (((1,), (0,)), ((), ())),
                               preferred_element_type=_F32)


def _mm_t(a, b):
    return jax.lax.dot_general(a, b, (((0,), (0,)), ((), ())),
                               preferred_element_type=_F32)


def _sigmoid_pre(y):
    return 0.5 * jnp.tanh(y) + 0.5


def _fwd_kernel(si_ref, ei_ref, bd_ref,
                data_ref, ti_ref, fi_ref,
                embed_ref, wi_ref, wh_ref, b_ref, wbd_ref, wout_ref, bout_ref,
                out_ref):
    iota_l256 = jax.lax.broadcasted_iota(jnp.int32, (1, 4 * _H), 1)
    scale_row = jnp.where((iota_l256 >= 2 * _H) & (iota_l256 < 3 * _H),
                          jnp.full((1, 4 * _H), 1.0, _F32),
                          jnp.full((1, 4 * _H), 0.5, _F32))
    iota_s256 = jax.lax.broadcasted_iota(jnp.int32, (4 * _H, 1), 0)
    scale_col = jnp.where((iota_s256 >= 2 * _H) & (iota_s256 < 3 * _H),
                          jnp.full((4 * _H, 1), 1.0, _F32),
                          jnp.full((4 * _H, 1), 0.5, _F32))
    ew = _mm(embed_ref[...], wi_ref[0] * scale_row)
    wh0 = wh_ref[0] * scale_row
    w1 = jnp.concatenate([wi_ref[1], wh_ref[1]], axis=0) * scale_row
    b0 = b_ref[0] * scale_col
    b1 = b_ref[1] * scale_col
    iota_s2 = jax.lax.broadcasted_iota(jnp.int32, (2, 1), 0)
    pm_col = jnp.where(iota_s2 == 0, jnp.full((2, 1), 0.5, _F32),
                       jnp.full((2, 1), -0.5, _F32))
    wd = _mm(wbd_ref[...], pm_col)
    bd_b = (bd_ref[0] - bd_ref[1]) * 0.5

    iota_col = jax.lax.broadcasted_iota(jnp.int32, (_N, 1), 0)
    iota_row1 = jax.lax.broadcasted_iota(jnp.int32, (1, _N), 1)
    ips = [(iota_col == si_ref[g]).astype(_F32) for g in range(_G)]
    exit_row = jnp.concatenate(
        [iota_row1 == ei_ref[g] for g in range(_G)], axis=1)
    exit_cols = [(iota_col == ei_ref[g]).astype(_F32) for g in range(_G)]

    iota_vcol = jax.lax.broadcasted_iota(jnp.int32, (_VOCAB, _OHC), 0)
    gx0 = []
    for tok in range(_L):
        chunks = []
        for cs in range(0, _W, _OHC):
            drow = data_ref[tok:tok + 1, cs:cs + _OHC]
            oh = (iota_vcol == drow).astype(_F32)
            chunks.append(_mm_t(ew, oh))
        gx0.append(jnp.concatenate(chunks, axis=1))

    iota_row = jax.lax.broadcasted_iota(jnp.int32, (_N, _N), 1)
    t_oh = [(ti_ref[g] == iota_row).astype(_F32) for g in range(_G)]
    f_oh = [(fi_ref[g] == iota_row).astype(_F32) for g in range(_G)]
    ones_col = jnp.ones((_N, 1), _F32)
    ones_rows = jnp.ones((8, _N), _F32)

    s_prev = jnp.zeros((4 * _H, _W), _F32)

    for _step in range(_STEPS):
        c0 = s_prev[0 * _H:1 * _H]
        c1 = s_prev[1 * _H:2 * _H]
        h0 = s_prev[2 * _H:3 * _H]
        h1 = s_prev[3 * _H:4 * _H]
        for tok in range(_L):
            g0 = gx0[tok] + _mm_t(wh0, h0) + b0
            ig = _sigmoid_pre(g0[0 * _H:1 * _H])
            fg = _sigmoid_pre(g0[1 * _H:2 * _H])
            gg = jnp.tanh(g0[2 * _H:3 * _H])
            og = _sigmoid_pre(g0[3 * _H:4 * _H])
            c0 = fg * c0 + ig * gg
            h0 = og * jnp.tanh(c0)
            inp1 = jnp.concatenate([h0, h1], axis=0)
            g1 = _mm_t(w1, inp1) + b1
            ig = _sigmoid_pre(g1[0 * _H:1 * _H])
            fg = _sigmoid_pre(g1[1 * _H:2 * _H])
            gg = jnp.tanh(g1[2 * _H:3 * _H])
            og = _sigmoid_pre(g1[3 * _H:4 * _H])
            c1 = fg * c1 + ig * gg
            h1 = og * jnp.tanh(c1)
        s_new = jnp.concatenate([c0, c1, h0, h1], axis=0)
        sm = jnp.where(exit_row, s_prev, s_new)

        s_parts = []
        for g in range(_G):
            sm_g = sm[:, g * _N:(g + 1) * _N]
            d = _mm_t(sm_g, wd) + bd_b
            pt = _sigmoid_pre(d) * ips[g]
            pf = ips[g] - pt
            bm = t_oh[g] * pt + f_oh[g] * pf
            sm_aug = jnp.concatenate([sm_g, ones_rows], axis=0)
            agg = _mm(sm_aug, bm)
            inv = 1.0 / (agg[4 * _H:4 * _H + 1] + 1e-7)
            s_parts.append(agg[0:4 * _H] * inv)
            ips[g] = _mm_t(bm, ones_col)
        s_prev = jnp.concatenate(s_parts, axis=1)

    es = jnp.concatenate(
        [_mm(s_prev[:, g * _N:(g + 1) * _N], exit_cols[g]) for g in range(_G)],
        axis=1)
    out = _mm_t(es, wout_ref[...]) + bout_ref[...]
    out_ref[...] = out


def kernel(data, true_branch_nodes, false_branch_nodes, start_index, exit_index,
           steps, embed, Wi, Wh, b, W_bd, b_bd, W_out, b_out):
    del steps
    dm = jnp.transpose(data, (2, 0, 1)).reshape(_L, _B * _N)
    ti3 = true_branch_nodes.reshape(_B, _N, 1)
    fi3 = false_branch_nodes.reshape(_B, _N, 1)
    b3 = b.reshape(_LAYERS, 4 * _H, 1)
    bout2 = b_out.reshape(1, _OUT)

    out = pl.pallas_call(
        _fwd_kernel,
        in_specs=[
            pl.BlockSpec(memory_space=pltpu.SMEM),
            pl.BlockSpec(memory_space=pltpu.SMEM),
            pl.BlockSpec(memory_space=pltpu.SMEM),
            pl.BlockSpec((_L, _W), lambda: (0, 0)),
            pl.BlockSpec((_G, _N, 1), lambda: (0, 0, 0)),
            pl.BlockSpec((_G, _N, 1), lambda: (0, 0, 0)),
            pl.BlockSpec((_VOCAB, _H), lambda: (0, 0)),
            pl.BlockSpec((_LAYERS, _H, 4 * _H), lambda: (0, 0, 0)),
            pl.BlockSpec((_LAYERS, _H, 4 * _H), lambda: (0, 0, 0)),
            pl.BlockSpec((_LAYERS, 4 * _H, 1), lambda: (0, 0, 0)),
            pl.BlockSpec((4 * _H, 2), lambda: (0, 0)),
            pl.BlockSpec((4 * _H, _OUT), lambda: (0, 0)),
            pl.BlockSpec((1, _OUT), lambda: (0, 0)),
        ],
        out_specs=pl.BlockSpec((_G, _OUT), lambda: (0, 0)),
        out_shape=jax.ShapeDtypeStruct((_G, _OUT), _F32),
    )(start_index, exit_index, b_bd, dm, ti3, fi3, embed, Wi, Wh, b3, W_bd,
      W_out, bout2)
    return out
